# baseline (speedup 1.0000x reference)
import functools

import jax
import jax.numpy as jnp
from jax import lax
from jax.experimental import pallas as pl
from jax.experimental.pallas import tpu as pltpu
from jax.experimental.pallas import tpu_sc as plsc

RPAD = 56
AGRP = 2
NBUF = 4
SKEW = 2
NC = 2
NS = 16
NW = NC * NS


@functools.lru_cache(maxsize=None)
def _build_gather(A, R, V, d):
    a_per_w = A // NW
    ngrp = a_per_w // AGRP
    nloop = ngrp // NBUF
    assert a_per_w * NW == A and ngrp * AGRP == a_per_w and nloop * NBUF == ngrp
    assert R <= RPAD and RPAD % 8 == 0 and AGRP * RPAD <= 128
    mesh = plsc.VectorSubcoreMesh(core_axis_name="c", subcore_axis_name="s")

    @functools.partial(
        pl.kernel,
        mesh=mesh,
        out_type=jax.ShapeDtypeStruct((A, R, d), jnp.float32),
        scratch_types=[
            pltpu.VMEM((a_per_w * RPAD,), jnp.int32),
            pltpu.VMEM((NBUF, AGRP * RPAD, d), jnp.float32),
        ] + [pltpu.SemaphoreType.DMA] * (2 * NBUF),
    )
    def k(table_hbm, idx_hbm, out_hbm, idx_v, rows_v, *sems):
        sems_g, sems_o = sems[:NBUF], sems[NBUF:]
        wid = lax.axis_index("s") * NC + lax.axis_index("c")
        abase = wid * a_per_w
        pltpu.sync_copy(idx_hbm.at[pl.ds(abase * RPAD, a_per_w * RPAD)], idx_v)

        def gather_desc(g, b):
            return pltpu.make_async_copy(
                table_hbm.at[idx_v.at[pl.ds(g * AGRP * RPAD, AGRP * RPAD)]],
                rows_v.at[b], sems_g[b])

        def out_desc(g, b, j):
            return pltpu.make_async_copy(
                rows_v.at[b, pl.ds(j * RPAD, R)],
                out_hbm.at[abase + g * AGRP + j], sems_o[b])

        def drain(g, b):
            gather_desc(g, b).wait()
            for j in range(AGRP):
                out_desc(g, b, j).start()

        def reclaim(g, b):
            for j in range(AGRP):
                out_desc(g, b, j).wait()

        for b in range(NBUF):
            gather_desc(b, b).start()
            if b >= SKEW:
                drain(b - SKEW, b - SKEW)

        def loop(i, carry):
            for b in range(NBUF):
                g = i * NBUF + b
                reclaim(g - NBUF, b)
                gather_desc(g, b).start()
                drain(g - SKEW, (b - SKEW) % NBUF)
            return carry

        lax.fori_loop(1, nloop, loop, 0)

        for g in range(ngrp - SKEW, ngrp):
            drain(g, g % NBUF)
        for b in range(NBUF):
            reclaim((nloop - 1) * NBUF + b, b)

    return k


def kernel(input_tensor, indices):
    d = input_tensor.shape[-1]
    A, R = indices.shape
    idx = jnp.pad(indices.astype(jnp.int32), ((0, 0), (0, RPAD - R)))
    return _build_gather(A, R, input_tensor.shape[0], d)(
        input_tensor, idx.reshape(-1))


# speedup vs baseline: 3.4846x; 3.4846x over previous
import functools

import jax
import jax.numpy as jnp
from jax import lax
from jax.experimental import pallas as pl
from jax.experimental.pallas import tpu as pltpu
from jax.experimental.pallas import tpu_sc as plsc

CHUNK = 128
NBUF = 5
SKEW = 2
NC = 2
NS = 16
NW = NC * NS


@functools.lru_cache(maxsize=None)
def _build_gather(B, V, d):
    b_per_w = B // NW
    nchunk = b_per_w // CHUNK
    ngroups = nchunk // NBUF
    assert b_per_w * NW == B and nchunk * CHUNK == b_per_w
    assert ngroups * NBUF == nchunk
    mesh = plsc.VectorSubcoreMesh(core_axis_name="c", subcore_axis_name="s")

    @functools.partial(
        pl.kernel,
        mesh=mesh,
        out_type=jax.ShapeDtypeStruct((B, d), jnp.float32),
        scratch_types=[
            pltpu.VMEM((b_per_w,), jnp.int32),
            pltpu.VMEM((NBUF, CHUNK, d), jnp.float32),
        ] + [pltpu.SemaphoreType.DMA] * (2 * NBUF),
    )
    def k(table_hbm, idx_hbm, out_hbm, idx_v, rows_v, *sems):
        sems_g, sems_o = sems[:NBUF], sems[NBUF:]
        wid = lax.axis_index("s") * NC + lax.axis_index("c")
        base = wid * b_per_w
        pltpu.sync_copy(idx_hbm.at[pl.ds(base, b_per_w)], idx_v)

        def out_desc(c, b):
            return pltpu.make_async_copy(
                rows_v.at[b], out_hbm.at[pl.ds(base + c * CHUNK, CHUNK)], sems_o[b])

        def gather_desc(c, b):
            return pltpu.make_async_copy(
                table_hbm.at[idx_v.at[pl.ds(c * CHUNK, CHUNK)]],
                rows_v.at[b], sems_g[b])

        def drain(c, b):
            gather_desc(c, b).wait()
            out_desc(c, b).start()

        for b in range(NBUF):
            gather_desc(b, b).start()
            if b >= SKEW:
                drain(b - SKEW, b - SKEW)

        def group(g, carry):
            for b in range(NBUF):
                c = g * NBUF + b
                out_desc(c - NBUF, b).wait()
                gather_desc(c, b).start()
                drain(c - SKEW, (b - SKEW) % NBUF)
            return carry

        lax.fori_loop(1, ngroups, group, 0)

        for c in range(nchunk - SKEW, nchunk):
            drain(c, c % NBUF)
        for b in range(NBUF):
            out_desc((ngroups - 1) * NBUF + b, b).wait()

    return k


def kernel(input_tensor, indices):
    d = input_tensor.shape[-1]
    B = indices.size
    idx = indices.reshape(-1).astype(jnp.int32)
    out = _build_gather(B, input_tensor.shape[0], d)(input_tensor, idx)
    one = lax.optimization_barrier(jnp.float32(1))
    return out.reshape(indices.shape + (d,)) * one
